# baseline (speedup 1.0000x reference)
import jax
import jax.numpy as jnp
from jax.experimental import pallas as pl
from jax.experimental.pallas import tpu as pltpu

B, S, D = 2, 2048, 1024
H, DKV = 16, 64
DFF = 4096
E = 16
C = 256
LN_EPS = 1e-6
GATE_EPS = 1e-9
SECOND_THRESHOLD = 0.2
LOSS_COEF = 0.01
FCH = 1024
NF = DFF // FCH
QB = 512
NQ = S // QB
_VMEM_PARAMS = pltpu.CompilerParams(vmem_limit_bytes=63 * 1024 * 1024)


_PREC = jax.lax.Precision.DEFAULT


def _dot(a, b):
    return jax.lax.dot_general(a, b, (((1,), (0,)), ((), ())),
                               precision=_PREC,
                               preferred_element_type=jnp.float32)


def _dot_tn(a, b):
    return jax.lax.dot_general(a, b, (((0,), (0,)), ((), ())),
                               precision=_PREC,
                               preferred_element_type=jnp.float32)


def _dot_nt(a, b):
    return jax.lax.dot_general(a, b, (((1,), (1,)), ((), ())),
                               precision=_PREC,
                               preferred_element_type=jnp.float32)


def _ln_kernel(h_ref, ln_ref, xn_ref):
    x = h_ref[0]
    var = jnp.mean(x * x, axis=-1, keepdims=True)
    xn_ref[0] = (x * jax.lax.rsqrt(var + LN_EPS)) * ln_ref[...]


def _qkv_kernel(xn_ref, wq_ref, wk_ref, wv_ref, q_ref, k_ref, v_ref):
    xn = xn_ref[0]
    q_ref[0, 0] = _dot(xn, wq_ref[0])
    k_ref[0, 0] = _dot(xn, wk_ref[0])
    v_ref[0, 0] = _dot(xn, wv_ref[0])


def _attn_kernel(q_ref, k_ref, v_ref, wo_ref, h_ref, out_ref):
    hh = pl.program_id(2)
    s = _dot_nt(q_ref[0, 0], k_ref[0, 0])
    m = jnp.max(s, axis=-1, keepdims=True)
    p = jnp.exp(s - m)
    p = p / jnp.sum(p, axis=-1, keepdims=True)
    ctx = _dot(p, v_ref[0, 0])
    contrib = _dot(ctx, wo_ref[0])

    @pl.when(hh == 0)
    def _():
        out_ref[0] = h_ref[0] + contrib

    @pl.when(hh != 0)
    def _():
        out_ref[0] = out_ref[0] + contrib


def _gating_kernel(h_ref, ln_ref, wg_ref, probs_ref, xn_ref, rt_ref, loss_ref):
    x = h_ref[0]
    var = jnp.mean(x * x, axis=-1, keepdims=True)
    xn = (x * jax.lax.rsqrt(var + LN_EPS)) * ln_ref[...]
    xn_ref[0] = xn
    logits = _dot(xn, wg_ref[...])
    m = jnp.max(logits, axis=-1, keepdims=True)
    ex = jnp.exp(logits - m)
    raw = ex / jnp.sum(ex, axis=-1, keepdims=True)

    eio = jax.lax.broadcasted_iota(jnp.int32, (S, E), 1).astype(jnp.float32)
    gate1 = jnp.max(raw, axis=-1, keepdims=True)
    idx1 = jnp.min(jnp.where(raw == gate1, eio, float(E)),
                   axis=-1, keepdims=True)
    mask1 = (eio == idx1).astype(jnp.float32)
    raw2 = raw * (1.0 - mask1)
    gate2 = jnp.max(raw2, axis=-1, keepdims=True)
    idx2 = jnp.min(jnp.where(raw2 == gate2, eio, float(E)),
                   axis=-1, keepdims=True)
    mask2 = (eio == idx2).astype(jnp.float32)

    denom = gate1 + gate2 + GATE_EPS
    g1n = gate1 / denom
    g2n = gate2 / denom

    density1 = jnp.mean(mask1, axis=0, keepdims=True)
    proxy = jnp.mean(raw, axis=0, keepdims=True)
    loss_part = jnp.sum(proxy * density1) * (float(E * E) / float(B * E)) * LOSS_COEF

    keep2 = (probs_ref[0] < g2n / SECOND_THRESHOLD).astype(jnp.float32)
    mask2 = mask2 * keep2

    ri = jax.lax.broadcasted_iota(jnp.int32, (S, S), 0)
    ci = jax.lax.broadcasted_iota(jnp.int32, (S, S), 1)
    lst = (ci < ri).astype(jnp.float32)
    cum1 = jax.lax.dot_general(lst, mask1, (((1,), (0,)), ((), ())),
                               preferred_element_type=jnp.float32)
    pie1 = cum1 * mask1
    mask1c = mask1 * (pie1 < float(C)).astype(jnp.float32)
    count1 = jnp.sum(mask1c, axis=0, keepdims=True)
    flat1 = jnp.sum(mask1c, axis=1, keepdims=True)
    pos1 = jnp.sum(pie1, axis=1, keepdims=True)
    g1 = g1n * flat1

    cum2 = jax.lax.dot_general(lst, mask2, (((1,), (0,)), ((), ())),
                               preferred_element_type=jnp.float32) + count1
    pie2 = cum2 * mask2
    mask2c = mask2 * (pie2 < float(C)).astype(jnp.float32)
    flat2 = jnp.sum(mask2c, axis=1, keepdims=True)
    pos2 = jnp.sum(pie2, axis=1, keepdims=True)
    g2 = g2n * flat2

    pos1c = jnp.where(flat1 > 0.0, pos1, float(C))
    pos2c = jnp.where(flat2 > 0.0, pos2, float(C))

    pad = jnp.zeros((S, 2), jnp.float32)
    rt_ref[0] = jnp.concatenate([idx1, pos1c, g1, idx2, pos2c, g2, pad], axis=1)

    @pl.when(pl.program_id(0) == 0)
    def _():
        loss_ref[...] = jnp.full((1, 1), loss_part, jnp.float32)

    @pl.when(pl.program_id(0) != 0)
    def _():
        loss_ref[...] = loss_ref[...] + loss_part


def _moe_ffn_kernel(xn_ref, rt_ref, w1_ref, w2_ref, slots_ref, ein_ref):
    e = pl.program_id(1)
    f = pl.program_id(2)
    ef = e.astype(jnp.float32)

    @pl.when(f == 0)
    def _():
        rt = rt_ref[0]
        idx1 = rt[:, 0:1]
        pos1 = rt[:, 1:2]
        idx2 = rt[:, 3:4]
        pos2 = rt[:, 4:5]
        ci = jax.lax.broadcasted_iota(jnp.int32, (S, C), 1).astype(jnp.float32)
        m1 = jnp.logical_and(idx1 == ef, pos1 == ci).astype(jnp.float32)
        m2 = jnp.logical_and(idx2 == ef, pos2 == ci).astype(jnp.float32)
        ein_ref[...] = _dot_tn(m1 + m2, xn_ref[0])

    hid = jnp.maximum(_dot(ein_ref[...], w1_ref[0]), 0.0)
    part = _dot(hid, w2_ref[0])

    @pl.when(f == 0)
    def _():
        slots_ref[0, 0] = part

    @pl.when(f != 0)
    def _():
        slots_ref[0, 0] = slots_ref[0, 0] + part


def _combine_kernel(rt_ref, slots_ref, h_ref, out_ref):
    e = pl.program_id(1)
    ef = e.astype(jnp.float32)
    rt = rt_ref[0]
    idx1 = rt[:, 0:1]
    pos1 = rt[:, 1:2]
    g1 = rt[:, 2:3]
    idx2 = rt[:, 3:4]
    pos2 = rt[:, 4:5]
    g2 = rt[:, 5:6]
    ci = jax.lax.broadcasted_iota(jnp.int32, (S, C), 1).astype(jnp.float32)
    m1 = jnp.logical_and(idx1 == ef, pos1 == ci).astype(jnp.float32)
    m2 = jnp.logical_and(idx2 == ef, pos2 == ci).astype(jnp.float32)
    comb = g1 * m1 + g2 * m2
    contrib = _dot(comb, slots_ref[0, 0])

    @pl.when(e == 0)
    def _():
        out_ref[0] = h_ref[0] + contrib

    @pl.when(e != 0)
    def _():
        out_ref[0] = out_ref[0] + contrib


def kernel(hidden_states, ln0_w, wq, wk, wv, wo, ln1_w, w_gating, w1, w2):
    f32 = jnp.float32
    wq_r = wq.reshape(D, H, DKV).transpose(1, 0, 2)
    wk_r = wk.reshape(D, H, DKV).transpose(1, 0, 2)
    wv_r = wv.reshape(D, H, DKV).transpose(1, 0, 2)
    wo_r = wo.reshape(H, DKV, D)
    ln0 = ln0_w.reshape(1, D)
    ln1 = ln1_w.reshape(1, D)
    probs_t = jax.random.uniform(jax.random.key(42), (B, S),
                                 dtype=f32).reshape(B, S, 1)

    xn0 = pl.pallas_call(
        _ln_kernel,
        grid=(B,),
        in_specs=[
            pl.BlockSpec((1, S, D), lambda b: (b, 0, 0)),
            pl.BlockSpec((1, D), lambda b: (0, 0)),
        ],
        out_specs=pl.BlockSpec((1, S, D), lambda b: (b, 0, 0)),
        out_shape=jax.ShapeDtypeStruct((B, S, D), f32),
        compiler_params=_VMEM_PARAMS,
    )(hidden_states, ln0)

    q, k, v = pl.pallas_call(
        _qkv_kernel,
        grid=(B, NQ, H),
        in_specs=[
            pl.BlockSpec((1, QB, D), lambda b, qi, hh: (b, qi, 0)),
            pl.BlockSpec((1, D, DKV), lambda b, qi, hh: (hh, 0, 0)),
            pl.BlockSpec((1, D, DKV), lambda b, qi, hh: (hh, 0, 0)),
            pl.BlockSpec((1, D, DKV), lambda b, qi, hh: (hh, 0, 0)),
        ],
        out_specs=[
            pl.BlockSpec((1, 1, QB, DKV), lambda b, qi, hh: (b, hh, qi, 0)),
            pl.BlockSpec((1, 1, QB, DKV), lambda b, qi, hh: (b, hh, qi, 0)),
            pl.BlockSpec((1, 1, QB, DKV), lambda b, qi, hh: (b, hh, qi, 0)),
        ],
        out_shape=[
            jax.ShapeDtypeStruct((B, H, S, DKV), f32),
            jax.ShapeDtypeStruct((B, H, S, DKV), f32),
            jax.ShapeDtypeStruct((B, H, S, DKV), f32),
        ],
        compiler_params=_VMEM_PARAMS,
    )(xn0, wq_r, wk_r, wv_r)

    h = pl.pallas_call(
        _attn_kernel,
        grid=(B, NQ, H),
        in_specs=[
            pl.BlockSpec((1, 1, QB, DKV), lambda b, qi, hh: (b, hh, qi, 0)),
            pl.BlockSpec((1, 1, S, DKV), lambda b, qi, hh: (b, hh, 0, 0)),
            pl.BlockSpec((1, 1, S, DKV), lambda b, qi, hh: (b, hh, 0, 0)),
            pl.BlockSpec((1, DKV, D), lambda b, qi, hh: (hh, 0, 0)),
            pl.BlockSpec((1, QB, D), lambda b, qi, hh: (b, qi, 0)),
        ],
        out_specs=pl.BlockSpec((1, QB, D), lambda b, qi, hh: (b, qi, 0)),
        out_shape=jax.ShapeDtypeStruct((B, S, D), f32),
        compiler_params=_VMEM_PARAMS,
    )(q, k, v, wo_r, hidden_states)

    xn, rt, loss = pl.pallas_call(
        _gating_kernel,
        grid=(B,),
        in_specs=[
            pl.BlockSpec((1, S, D), lambda b: (b, 0, 0)),
            pl.BlockSpec((1, D), lambda b: (0, 0)),
            pl.BlockSpec((D, E), lambda b: (0, 0)),
            pl.BlockSpec((1, S, 1), lambda b: (b, 0, 0)),
        ],
        out_specs=[
            pl.BlockSpec((1, S, D), lambda b: (b, 0, 0)),
            pl.BlockSpec((1, S, 8), lambda b: (b, 0, 0)),
            pl.BlockSpec((1, 1), lambda b: (0, 0)),
        ],
        out_shape=[
            jax.ShapeDtypeStruct((B, S, D), f32),
            jax.ShapeDtypeStruct((B, S, 8), f32),
            jax.ShapeDtypeStruct((1, 1), f32),
        ],
        compiler_params=_VMEM_PARAMS,
    )(h, ln1, w_gating, probs_t)

    slots = pl.pallas_call(
        _moe_ffn_kernel,
        grid=(B, E, NF),
        in_specs=[
            pl.BlockSpec((1, S, D), lambda b, e, f: (b, 0, 0)),
            pl.BlockSpec((1, S, 8), lambda b, e, f: (b, 0, 0)),
            pl.BlockSpec((1, D, FCH), lambda b, e, f: (e, 0, f)),
            pl.BlockSpec((1, FCH, D), lambda b, e, f: (e, f, 0)),
        ],
        out_specs=pl.BlockSpec((1, 1, C, D), lambda b, e, f: (b, e, 0, 0)),
        out_shape=jax.ShapeDtypeStruct((B, E, C, D), f32),
        scratch_shapes=[pltpu.VMEM((C, D), f32)],
        compiler_params=_VMEM_PARAMS,
    )(xn, rt, w1, w2)

    out = pl.pallas_call(
        _combine_kernel,
        grid=(B, E),
        in_specs=[
            pl.BlockSpec((1, S, 8), lambda b, e: (b, 0, 0)),
            pl.BlockSpec((1, 1, C, D), lambda b, e: (b, e, 0, 0)),
            pl.BlockSpec((1, S, D), lambda b, e: (b, 0, 0)),
        ],
        out_specs=pl.BlockSpec((1, S, D), lambda b, e: (b, 0, 0)),
        out_shape=jax.ShapeDtypeStruct((B, S, D), f32),
        compiler_params=_VMEM_PARAMS,
    )(rt, slots, h)

    return (out, out[1:], loss[0, 0])
